# baseline (speedup 1.0000x reference)
import functools

import jax
import jax.numpy as jnp
from jax import lax
from jax.experimental import pallas as pl
from jax.experimental.pallas import tpu as pltpu
from jax.experimental.pallas import tpu_sc as plsc

F32 = jnp.float32
D = 64
EP = 819200
NACC = 50016
_NC, _NS = 2, 16


def _acc_stats(st_ref, z, valid):
    @pl.when(pl.program_id(0) == 0)
    def _():
        st_ref[...] = jnp.zeros_like(st_ref)
    if valid is not None:
        bm = z.shape[0]
        row = (pl.program_id(0) * bm
               + lax.broadcasted_iota(jnp.int32, (bm, 1), 0))
        z = jnp.where(row < valid, z, 0.0)
    upd = jnp.concatenate(
        [jnp.sum(z, axis=0)[None, :], jnp.sum(z * z, axis=0)[None, :],
         jnp.zeros((6, z.shape[1]), F32)], axis=0)
    st_ref[...] += upd


def _proj3_body(x_ref, w_ref, b_ref, wi_ref, wj_ref, h_ref, ha_ref, hb_ref):
    h = jnp.dot(x_ref[...], w_ref[...],
                preferred_element_type=F32) + b_ref[...][0][None, :]
    h_ref[...] = h
    ha_ref[...] = jnp.dot(h, wi_ref[...], preferred_element_type=F32)
    hb_ref[...] = jnp.dot(h, wj_ref[...], preferred_element_type=F32)


def _passA_body(valid, ga_ref, gb_ref, e_ref, w_ref, z_ref, st_ref):
    z = ga_ref[...] + gb_ref[...] + jnp.dot(e_ref[...], w_ref[...],
                                            preferred_element_type=F32)
    z_ref[...] = z
    _acc_stats(st_ref, z, valid)


def _passA0_body(valid, ga_ref, gb_ref, ea_ref, v_ref, z_ref, st_ref):
    z = ga_ref[...] + gb_ref[...] + ea_ref[...] * v_ref[...]
    z_ref[...] = z
    _acc_stats(st_ref, z, valid)


def _armm_body(valid, z_ref, a_ref, c_ref, w_ref, o_ref, st_ref):
    m = jnp.maximum(z_ref[...] * a_ref[...][0][None, :] + c_ref[...][0][None, :], 0.0)
    z2 = jnp.dot(m, w_ref[...], preferred_element_type=F32)
    o_ref[...] = z2
    _acc_stats(st_ref, z2, valid)


def _armm_split_body(valid, z_ref, a_ref, c_ref, w_ref, oh_ref, st_ref):
    m = jnp.maximum(z_ref[...] * a_ref[...][0][None, :] + c_ref[...][0][None, :], 0.0)
    z2 = jnp.dot(m, w_ref[...], preferred_element_type=F32)
    oh_ref[0, :, :] = z2[:, :32]
    oh_ref[1, :, :] = z2[:, 32:]
    _acc_stats(st_ref, z2, valid)


def _passCh_body(zh_ref, a_ref, c_ref, p_ref, enew_ref):
    z = jnp.concatenate([zh_ref[0], zh_ref[1]], axis=1)
    m2 = jnp.maximum(z * a_ref[...][0][None, :] + c_ref[...][0][None, :], 0.0)
    enew_ref[...] = p_ref[...] + m2


def _passCh0_body(zh_ref, a_ref, c_ref, ea_ref, wv_ref, bv_ref, enew_ref):
    z = jnp.concatenate([zh_ref[0], zh_ref[1]], axis=1)
    m2 = jnp.maximum(z * a_ref[...][0][None, :] + c_ref[...][0][None, :], 0.0)
    enew_ref[...] = ea_ref[...] * wv_ref[...] + bv_ref[...] + m2


def _passCP_body(z_ref, a_ref, c_ref, p_ref, w_ref, b_ref, o_ref):
    m2 = jnp.maximum(z_ref[...] * a_ref[...][0][None, :] + c_ref[...][0][None, :], 0.0)
    s = jnp.sum((p_ref[...] + m2) * w_ref[...][0][None, :], axis=1) + b_ref[...][0, 0]
    o_ref[...] = s.reshape(o_ref.shape)


def _nodeP1_body(h_ref, agg_ref, uh_ref, ua0_ref, ua1_ref, z_ref, st_ref):
    z = (jnp.dot(h_ref[...], uh_ref[...], preferred_element_type=F32)
         + jnp.dot(agg_ref[0], ua0_ref[...], preferred_element_type=F32)
         + jnp.dot(agg_ref[1], ua1_ref[...], preferred_element_type=F32))
    z_ref[...] = z
    _acc_stats(st_ref, z, None)


def _resid3_body(z_ref, a_ref, c_ref, h_ref, wi_ref, wj_ref,
                 o_ref, ha_ref, hb_ref):
    hn = h_ref[...] + jnp.maximum(
        z_ref[...] * a_ref[...][0][None, :] + c_ref[...][0][None, :], 0.0)
    o_ref[...] = hn
    ha_ref[...] = jnp.dot(hn, wi_ref[...], preferred_element_type=F32)
    hb_ref[...] = jnp.dot(hn, wj_ref[...], preferred_element_type=F32)


def _rows_block(rows):
    for bm in (4096, 5000):
        if rows % bm == 0:
            return bm
    return rows


def _mat_spec(bm, ncols):
    return pl.BlockSpec((bm, ncols), lambda i: (i, 0))


def _fix_spec(shape):
    nd = len(shape)
    return pl.BlockSpec(shape, lambda i: (0,) * nd)


def _half_spec(bm):
    return pl.BlockSpec((2, bm, 32), lambda i: (0, i, 0))


_ST_SPEC = pl.BlockSpec((8, D), lambda i: (0, 0))
_ST_SHAPE = jax.ShapeDtypeStruct((8, D), F32)


def _proj3(x, w, b, wi, wj):
    rows, k = x.shape
    bm = _rows_block(rows)
    return pl.pallas_call(
        _proj3_body,
        grid=(rows // bm,),
        in_specs=[_mat_spec(bm, k), _fix_spec((k, D)), _fix_spec((1, D)),
                  _fix_spec((D, D)), _fix_spec((D, D))],
        out_specs=[_mat_spec(bm, D)] * 3,
        out_shape=[jax.ShapeDtypeStruct((rows, D), F32)] * 3,
    )(x, w, b.reshape(1, D), wi, wj)


def _passA(ga, gb, e, w, rows, valid):
    bm = _rows_block(rows)
    return pl.pallas_call(
        functools.partial(_passA_body, valid),
        grid=(rows // bm,),
        in_specs=[_mat_spec(bm, D)] * 3 + [_fix_spec((D, D))],
        out_specs=[_mat_spec(bm, D), _ST_SPEC],
        out_shape=[jax.ShapeDtypeStruct((rows, D), F32), _ST_SHAPE],
    )(ga, gb, e, w)


def _passA0(ga, gb, ea2d, v, rows, valid):
    bm = _rows_block(rows)
    return pl.pallas_call(
        functools.partial(_passA0_body, valid),
        grid=(rows // bm,),
        in_specs=[_mat_spec(bm, D), _mat_spec(bm, D),
                  _mat_spec(bm, 1), _fix_spec((1, D))],
        out_specs=[_mat_spec(bm, D), _ST_SPEC],
        out_shape=[jax.ShapeDtypeStruct((rows, D), F32), _ST_SHAPE],
    )(ga, gb, ea2d, v)


def _armm(z, a, c, w, valid=None):
    rows = z.shape[0]
    bm = _rows_block(rows)
    return pl.pallas_call(
        functools.partial(_armm_body, valid),
        grid=(rows // bm,),
        in_specs=[_mat_spec(bm, D), _fix_spec((1, D)), _fix_spec((1, D)),
                  _fix_spec((D, D))],
        out_specs=[_mat_spec(bm, D), _ST_SPEC],
        out_shape=[jax.ShapeDtypeStruct((rows, D), F32), _ST_SHAPE],
    )(z, a, c, w)


def _armm_split(z, a, c, w, valid):
    rows = z.shape[0]
    bm = _rows_block(rows)
    return pl.pallas_call(
        functools.partial(_armm_split_body, valid),
        grid=(rows // bm,),
        in_specs=[_mat_spec(bm, D), _fix_spec((1, D)), _fix_spec((1, D)),
                  _fix_spec((D, D))],
        out_specs=[_half_spec(bm), _ST_SPEC],
        out_shape=[jax.ShapeDtypeStruct((2, rows, 32), F32), _ST_SHAPE],
    )(z, a, c, w)


def _passCh(zh, a, c, prev):
    rows = prev.shape[0]
    bm = _rows_block(rows)
    return pl.pallas_call(
        _passCh_body,
        grid=(rows // bm,),
        in_specs=[_half_spec(bm), _fix_spec((1, D)), _fix_spec((1, D)),
                  _mat_spec(bm, D)],
        out_specs=_mat_spec(bm, D),
        out_shape=jax.ShapeDtypeStruct((rows, D), F32),
    )(zh, a, c, prev)


def _passCh0(zh, a, c, ea2d, wv, bv):
    rows = zh.shape[1]
    bm = _rows_block(rows)
    return pl.pallas_call(
        _passCh0_body,
        grid=(rows // bm,),
        in_specs=[_half_spec(bm), _fix_spec((1, D)), _fix_spec((1, D)),
                  _mat_spec(bm, 1), _fix_spec((1, D)), _fix_spec((1, D))],
        out_specs=_mat_spec(bm, D),
        out_shape=jax.ShapeDtypeStruct((rows, D), F32),
    )(zh, a, c, ea2d, wv, bv)


def _passCP(z, a, c, prev, wt, b):
    rows = z.shape[0]
    bm = _rows_block(rows)
    return pl.pallas_call(
        _passCP_body,
        grid=(rows // bm,),
        in_specs=[_mat_spec(bm, D), _fix_spec((1, D)), _fix_spec((1, D)),
                  _mat_spec(bm, D), _fix_spec((1, D)), _fix_spec((1, 1))],
        out_specs=_mat_spec(bm // 128, 128),
        out_shape=jax.ShapeDtypeStruct((rows // 128, 128), F32),
    )(z, a, c, prev, wt, b.reshape(1, 1))


def _nodeP1(h, aggh, uh, ua0, ua1):
    rows = h.shape[0]
    bm = _rows_block(rows)
    return pl.pallas_call(
        _nodeP1_body,
        grid=(rows // bm,),
        in_specs=[_mat_spec(bm, D), _half_spec(bm),
                  _fix_spec((D, D)), _fix_spec((32, D)), _fix_spec((32, D))],
        out_specs=[_mat_spec(bm, D), _ST_SPEC],
        out_shape=[jax.ShapeDtypeStruct((rows, D), F32), _ST_SHAPE],
    )(h, aggh, uh, ua0, ua1)


def _resid3(z, a, c, h, wi, wj):
    rows = h.shape[0]
    bm = _rows_block(rows)
    return pl.pallas_call(
        _resid3_body,
        grid=(rows // bm,),
        in_specs=[_mat_spec(bm, D), _fix_spec((1, D)), _fix_spec((1, D)),
                  _mat_spec(bm, D), _fix_spec((D, D)), _fix_spec((D, D))],
        out_specs=[_mat_spec(bm, D)] * 3,
        out_shape=[jax.ShapeDtypeStruct((rows, D), F32)] * 3,
    )(z, a, c, h, wi, wj)


def _affine(st, count, g, bb):
    mu = st[0] / count
    var = st[1] / count - mu * mu
    a = g * lax.rsqrt(var + 1e-5)
    c = bb - mu * a
    return a.reshape(1, D), c.reshape(1, D)


def _sc_gather2_body(nchunks, taba_hbm, tabb_hbm, idxa_hbm, idxb_hbm,
                     outa_hbm, outb_hbm, idxa_v, idxb_v, bufa, bufb, gsa, gsb):
    wid = lax.axis_index("s") * _NC + lax.axis_index("c")
    nrows = 8 * nchunks
    row0 = wid * nrows
    pltpu.sync_copy(idxa_hbm.at[pl.ds(row0, nrows)], idxa_v)
    pltpu.sync_copy(idxb_hbm.at[pl.ds(row0, nrows)], idxb_v)

    def body(g, carry):
        ra = g * 4
        cpa = [pltpu.async_copy(taba_hbm.at[idxa_v.at[ra + j]],
                                bufa.at[pl.ds(j * 128, 128)], gsa)
               for j in range(4)]
        cpb = [pltpu.async_copy(tabb_hbm.at[idxb_v.at[ra + j]],
                                bufb.at[pl.ds(j * 128, 128)], gsb)
               for j in range(4)]
        for cp in cpa:
            cp.wait()
        pltpu.sync_copy(bufa, outa_hbm.at[pl.ds((row0 + ra) * 128, 512)])
        for cp in cpb:
            cp.wait()
        pltpu.sync_copy(bufb, outb_hbm.at[pl.ds((row0 + ra) * 128, 512)])
        return carry

    lax.fori_loop(0, 2 * nchunks, body, 0)


def _sc_gather2(taba, tabb, idxa2d, idxb2d, ep):
    nchunks = ep // (1024 * _NC * _NS)
    mesh = plsc.VectorSubcoreMesh(core_axis_name="c", subcore_axis_name="s")
    return pl.kernel(
        functools.partial(_sc_gather2_body, nchunks),
        mesh=mesh,
        out_type=[jax.ShapeDtypeStruct((ep, D), F32)] * 2,
        scratch_types=[pltpu.VMEM((8 * nchunks, 128), jnp.int32),
                       pltpu.VMEM((8 * nchunks, 128), jnp.int32),
                       pltpu.VMEM((512, D), F32),
                       pltpu.VMEM((512, D), F32),
                       pltpu.SemaphoreType.DMA,
                       pltpu.SemaphoreType.DMA],
        compiler_params=pltpu.CompilerParams(use_tc_tiling_on_sc=False,
                                            skip_device_barrier=True),
    )(taba, tabb, idxa2d, idxb2d)


def _sc_scatter_body(nchunks, n_out, zh_hbm, idxs_hbm, zeros_hbm, ac_hbm,
                     out_hbm, idx_v, vals_v, ac_v, acc_sh):
    c = lax.axis_index("c")
    t = lax.axis_index("s")
    zrows = NACC // _NS
    pltpu.sync_copy(zeros_hbm.at[pl.ds(t * zrows, zrows)],
                    acc_sh.at[pl.ds(t * zrows, zrows)])
    pltpu.sync_copy(ac_hbm.at[0, c], ac_v.at[0])
    pltpu.sync_copy(ac_hbm.at[1, c], ac_v.at[1])
    a0 = ac_v[0, pl.ds(0, 16)]
    a1 = ac_v[0, pl.ds(16, 16)]
    c0 = ac_v[1, pl.ds(0, 16)]
    c1 = ac_v[1, pl.ds(16, 16)]
    plsc.subcore_barrier()
    row0 = t * (4 * nchunks)

    def body(s, carry):
        r0 = row0 + s * 4
        pltpu.sync_copy(idxs_hbm.at[pl.ds(r0, 4)], idx_v)
        pltpu.sync_copy(zh_hbm.at[c, pl.ds(r0 * 128, 512)], vals_v)

        def inner(r, cy):
            vals_v[r, pl.ds(0, 16)] = jnp.maximum(
                vals_v[r, pl.ds(0, 16)] * a0 + c0, 0.0)
            vals_v[r, pl.ds(16, 16)] = jnp.maximum(
                vals_v[r, pl.ds(16, 16)] * a1 + c1, 0.0)
            return cy

        lax.fori_loop(0, 512, inner, 0)
        for j in range(4):
            pltpu.sync_copy(vals_v.at[pl.ds(j * 128, 128)],
                            acc_sh.at[idx_v.at[j]], add=True)
        return carry

    lax.fori_loop(0, nchunks, body, 0)
    plsc.subcore_barrier()
    orow = n_out // _NS
    pltpu.sync_copy(acc_sh.at[pl.ds(t * orow, orow)],
                    out_hbm.at[c, pl.ds(t * orow, orow)])


def _sc_scatter(zh, idxs2d, zeros_acc, ac, n_out):
    ep = zh.shape[1]
    nchunks = ep // (512 * _NS)
    mesh = plsc.VectorSubcoreMesh(core_axis_name="c", subcore_axis_name="s")
    return pl.kernel(
        functools.partial(_sc_scatter_body, nchunks, n_out),
        mesh=mesh,
        out_type=jax.ShapeDtypeStruct((2, n_out, 32), F32),
        scratch_types=[pltpu.VMEM((4, 128), jnp.int32),
                       pltpu.VMEM((512, 32), F32),
                       pltpu.VMEM((2, 32), F32),
                       pltpu.VMEM_SHARED((NACC, 32), F32)],
        compiler_params=pltpu.CompilerParams(use_tc_tiling_on_sc=False,
                                            skip_device_barrier=True),
    )(zh, idxs2d, zeros_acc, ac)


def kernel(x, edge_attr, edge_index, w_in, b_in, w_e, b_e,
           msg_w1, msg_b1, msg_g1, msg_bb1, msg_w2, msg_b2, msg_g2, msg_bb2,
           upd_w1, upd_b1, upd_g1, upd_bb1, upd_w2, upd_b2, upd_g2, upd_bb2,
           w_pred, b_pred):
    src = edge_index[0]
    dst = edge_index[1]
    n = x.shape[0]
    e_cnt = src.shape[0]
    pad = EP - e_cnt

    dst_g = jnp.concatenate([dst, jnp.zeros((pad,), jnp.int32)]).reshape(EP // 128, 128)
    src_g = jnp.concatenate([src, jnp.zeros((pad,), jnp.int32)]).reshape(EP // 128, 128)
    dst_s = jnp.concatenate([dst, jnp.full((pad,), n, jnp.int32)]).reshape(EP // 128, 128)
    zeros_acc = jnp.zeros((NACC, 32), F32)
    ea2d = jnp.concatenate([edge_attr,
                            jnp.zeros((pad,), F32)]).reshape(EP, 1)

    h, ha, hb = _proj3(x, w_in, b_in, msg_w1[0, :D], msg_w1[0, D:2 * D])
    e = None
    out = None
    for l in range(3):
        we = msg_w1[l, 2 * D:]
        ga, gb = _sc_gather2(ha, hb, dst_g, src_g, EP)
        if l == 0:
            v = w_e @ we
            z1, st1 = _passA0(ga, gb, ea2d, v, EP, e_cnt)
        else:
            z1, st1 = _passA(ga, gb, e, we, EP, e_cnt)
        a1, c1 = _affine(st1, e_cnt, msg_g1[l], msg_bb1[l])
        if l == 2:
            z2, st2 = _armm(z1, a1, c1, msg_w2[l], e_cnt)
            a2, c2 = _affine(st2, e_cnt, msg_g2[l], msg_bb2[l])
            out = _passCP(z2, a2, c2, e, w_pred.reshape(1, D), b_pred)
            break
        z2h, st2 = _armm_split(z1, a1, c1, msg_w2[l], e_cnt)
        a2, c2 = _affine(st2, e_cnt, msg_g2[l], msg_bb2[l])
        ac = jnp.concatenate([a2.reshape(1, 2, 32), c2.reshape(1, 2, 32)],
                             axis=0)
        aggh = _sc_scatter(z2h, dst_s, zeros_acc, ac, n)
        if l == 0:
            e = _passCh0(z2h, a2, c2, ea2d, w_e, b_e.reshape(1, D))
        else:
            e = _passCh(z2h, a2, c2, e)
        zu1, su1 = _nodeP1(h, aggh, upd_w1[l, :D], upd_w1[l, D:D + 32],
                           upd_w1[l, D + 32:])
        au1, cu1 = _affine(su1, n, upd_g1[l], upd_bb1[l])
        zu2, su2 = _armm(zu1, au1, cu1, upd_w2[l])
        au2, cu2 = _affine(su2, n, upd_g2[l], upd_bb2[l])
        h, ha, hb = _resid3(zu2, au2, cu2, h, msg_w1[l + 1, :D],
                            msg_w1[l + 1, D:2 * D])
    return out.reshape(-1)[:e_cnt, None]


# speedup vs baseline: 1.0002x; 1.0002x over previous
import functools

import jax
import jax.numpy as jnp
from jax import lax
from jax.experimental import pallas as pl
from jax.experimental.pallas import tpu as pltpu
from jax.experimental.pallas import tpu_sc as plsc

F32 = jnp.float32
D = 64
EP = 819200
NACC = 50016
_NC, _NS = 2, 16


def _acc_stats(st_ref, z, valid):
    @pl.when(pl.program_id(0) == 0)
    def _():
        st_ref[...] = jnp.zeros_like(st_ref)
    if valid is not None:
        bm = z.shape[0]
        row = (pl.program_id(0) * bm
               + lax.broadcasted_iota(jnp.int32, (bm, 1), 0))
        z = jnp.where(row < valid, z, 0.0)
    upd = jnp.concatenate(
        [jnp.sum(z, axis=0)[None, :], jnp.sum(z * z, axis=0)[None, :],
         jnp.zeros((6, z.shape[1]), F32)], axis=0)
    st_ref[...] += upd


def _proj3_body(x_ref, w_ref, b_ref, wi_ref, wj_ref, h_ref, ha_ref, hb_ref):
    h = jnp.dot(x_ref[...], w_ref[...],
                preferred_element_type=F32) + b_ref[...][0][None, :]
    h_ref[...] = h
    ha_ref[...] = jnp.dot(h, wi_ref[...], preferred_element_type=F32)
    hb_ref[...] = jnp.dot(h, wj_ref[...], preferred_element_type=F32)


def _passA_body(valid, ga_ref, gb_ref, e_ref, w_ref, z_ref, st_ref):
    z = ga_ref[...] + gb_ref[...] + jnp.dot(e_ref[...], w_ref[...],
                                            preferred_element_type=F32)
    z_ref[...] = z
    _acc_stats(st_ref, z, valid)


def _passA0_body(valid, ga_ref, gb_ref, ea_ref, v_ref, z_ref, st_ref):
    z = ga_ref[...] + gb_ref[...] + ea_ref[...] * v_ref[...]
    z_ref[...] = z
    _acc_stats(st_ref, z, valid)


def _armm_body(valid, z_ref, a_ref, c_ref, w_ref, o_ref, st_ref):
    m = jnp.maximum(z_ref[...] * a_ref[...][0][None, :] + c_ref[...][0][None, :], 0.0)
    z2 = jnp.dot(m, w_ref[...], preferred_element_type=F32)
    o_ref[...] = z2
    _acc_stats(st_ref, z2, valid)


def _armm_split_body(valid, z_ref, a_ref, c_ref, w_ref, oh_ref, st_ref):
    m = jnp.maximum(z_ref[...] * a_ref[...][0][None, :] + c_ref[...][0][None, :], 0.0)
    z2 = jnp.dot(m, w_ref[...], preferred_element_type=F32)
    oh_ref[0, :, :] = z2[:, :32]
    oh_ref[1, :, :] = z2[:, 32:]
    _acc_stats(st_ref, z2, valid)


def _passCh_body(zh_ref, a_ref, c_ref, p_ref, enew_ref):
    z = jnp.concatenate([zh_ref[0], zh_ref[1]], axis=1)
    m2 = jnp.maximum(z * a_ref[...][0][None, :] + c_ref[...][0][None, :], 0.0)
    enew_ref[...] = p_ref[...] + m2


def _passCh0_body(zh_ref, a_ref, c_ref, ea_ref, wv_ref, bv_ref, enew_ref):
    z = jnp.concatenate([zh_ref[0], zh_ref[1]], axis=1)
    m2 = jnp.maximum(z * a_ref[...][0][None, :] + c_ref[...][0][None, :], 0.0)
    enew_ref[...] = ea_ref[...] * wv_ref[...] + bv_ref[...] + m2


def _passCP_body(z_ref, a_ref, c_ref, p_ref, w_ref, b_ref, o_ref):
    m2 = jnp.maximum(z_ref[...] * a_ref[...][0][None, :] + c_ref[...][0][None, :], 0.0)
    s = jnp.sum((p_ref[...] + m2) * w_ref[...][0][None, :], axis=1) + b_ref[...][0, 0]
    o_ref[...] = s.reshape(o_ref.shape)


def _nodeP1_body(h_ref, agg_ref, uh_ref, ua0_ref, ua1_ref, z_ref, st_ref):
    z = (jnp.dot(h_ref[...], uh_ref[...], preferred_element_type=F32)
         + jnp.dot(agg_ref[0], ua0_ref[...], preferred_element_type=F32)
         + jnp.dot(agg_ref[1], ua1_ref[...], preferred_element_type=F32))
    z_ref[...] = z
    _acc_stats(st_ref, z, None)


def _resid3_body(z_ref, a_ref, c_ref, h_ref, wi_ref, wj_ref,
                 o_ref, ha_ref, hb_ref):
    hn = h_ref[...] + jnp.maximum(
        z_ref[...] * a_ref[...][0][None, :] + c_ref[...][0][None, :], 0.0)
    o_ref[...] = hn
    ha_ref[...] = jnp.dot(hn, wi_ref[...], preferred_element_type=F32)
    hb_ref[...] = jnp.dot(hn, wj_ref[...], preferred_element_type=F32)


def _rows_block(rows):
    for bm in (4096, 5000):
        if rows % bm == 0:
            return bm
    return rows


def _mat_spec(bm, ncols):
    return pl.BlockSpec((bm, ncols), lambda i: (i, 0))


def _fix_spec(shape):
    nd = len(shape)
    return pl.BlockSpec(shape, lambda i: (0,) * nd)


def _half_spec(bm):
    return pl.BlockSpec((2, bm, 32), lambda i: (0, i, 0))


_ST_SPEC = pl.BlockSpec((8, D), lambda i: (0, 0))
_ST_SHAPE = jax.ShapeDtypeStruct((8, D), F32)


def _proj3(x, w, b, wi, wj):
    rows, k = x.shape
    bm = _rows_block(rows)
    return pl.pallas_call(
        _proj3_body,
        grid=(rows // bm,),
        in_specs=[_mat_spec(bm, k), _fix_spec((k, D)), _fix_spec((1, D)),
                  _fix_spec((D, D)), _fix_spec((D, D))],
        out_specs=[_mat_spec(bm, D)] * 3,
        out_shape=[jax.ShapeDtypeStruct((rows, D), F32)] * 3,
    )(x, w, b.reshape(1, D), wi, wj)


def _passA(ga, gb, e, w, rows, valid):
    bm = _rows_block(rows)
    return pl.pallas_call(
        functools.partial(_passA_body, valid),
        grid=(rows // bm,),
        in_specs=[_mat_spec(bm, D)] * 3 + [_fix_spec((D, D))],
        out_specs=[_mat_spec(bm, D), _ST_SPEC],
        out_shape=[jax.ShapeDtypeStruct((rows, D), F32), _ST_SHAPE],
    )(ga, gb, e, w)


def _passA0(ga, gb, ea2d, v, rows, valid):
    bm = _rows_block(rows)
    return pl.pallas_call(
        functools.partial(_passA0_body, valid),
        grid=(rows // bm,),
        in_specs=[_mat_spec(bm, D), _mat_spec(bm, D),
                  _mat_spec(bm, 1), _fix_spec((1, D))],
        out_specs=[_mat_spec(bm, D), _ST_SPEC],
        out_shape=[jax.ShapeDtypeStruct((rows, D), F32), _ST_SHAPE],
    )(ga, gb, ea2d, v)


def _armm(z, a, c, w, valid=None):
    rows = z.shape[0]
    bm = _rows_block(rows)
    return pl.pallas_call(
        functools.partial(_armm_body, valid),
        grid=(rows // bm,),
        in_specs=[_mat_spec(bm, D), _fix_spec((1, D)), _fix_spec((1, D)),
                  _fix_spec((D, D))],
        out_specs=[_mat_spec(bm, D), _ST_SPEC],
        out_shape=[jax.ShapeDtypeStruct((rows, D), F32), _ST_SHAPE],
    )(z, a, c, w)


def _armm_split(z, a, c, w, valid):
    rows = z.shape[0]
    bm = _rows_block(rows)
    return pl.pallas_call(
        functools.partial(_armm_split_body, valid),
        grid=(rows // bm,),
        in_specs=[_mat_spec(bm, D), _fix_spec((1, D)), _fix_spec((1, D)),
                  _fix_spec((D, D))],
        out_specs=[_half_spec(bm), _ST_SPEC],
        out_shape=[jax.ShapeDtypeStruct((2, rows, 32), F32), _ST_SHAPE],
    )(z, a, c, w)


def _passCh(zh, a, c, prev):
    rows = prev.shape[0]
    bm = _rows_block(rows)
    return pl.pallas_call(
        _passCh_body,
        grid=(rows // bm,),
        in_specs=[_half_spec(bm), _fix_spec((1, D)), _fix_spec((1, D)),
                  _mat_spec(bm, D)],
        out_specs=_mat_spec(bm, D),
        out_shape=jax.ShapeDtypeStruct((rows, D), F32),
    )(zh, a, c, prev)


def _passCh0(zh, a, c, ea2d, wv, bv):
    rows = zh.shape[1]
    bm = _rows_block(rows)
    return pl.pallas_call(
        _passCh0_body,
        grid=(rows // bm,),
        in_specs=[_half_spec(bm), _fix_spec((1, D)), _fix_spec((1, D)),
                  _mat_spec(bm, 1), _fix_spec((1, D)), _fix_spec((1, D))],
        out_specs=_mat_spec(bm, D),
        out_shape=jax.ShapeDtypeStruct((rows, D), F32),
    )(zh, a, c, ea2d, wv, bv)


def _passCP(z, a, c, prev, wt, b):
    rows = z.shape[0]
    bm = _rows_block(rows)
    return pl.pallas_call(
        _passCP_body,
        grid=(rows // bm,),
        in_specs=[_mat_spec(bm, D), _fix_spec((1, D)), _fix_spec((1, D)),
                  _mat_spec(bm, D), _fix_spec((1, D)), _fix_spec((1, 1))],
        out_specs=_mat_spec(bm // 128, 128),
        out_shape=jax.ShapeDtypeStruct((rows // 128, 128), F32),
    )(z, a, c, prev, wt, b.reshape(1, 1))


def _nodeP1(h, aggh, uh, ua0, ua1):
    rows = h.shape[0]
    bm = _rows_block(rows)
    return pl.pallas_call(
        _nodeP1_body,
        grid=(rows // bm,),
        in_specs=[_mat_spec(bm, D), _half_spec(bm),
                  _fix_spec((D, D)), _fix_spec((32, D)), _fix_spec((32, D))],
        out_specs=[_mat_spec(bm, D), _ST_SPEC],
        out_shape=[jax.ShapeDtypeStruct((rows, D), F32), _ST_SHAPE],
    )(h, aggh, uh, ua0, ua1)


def _resid3(z, a, c, h, wi, wj):
    rows = h.shape[0]
    bm = _rows_block(rows)
    return pl.pallas_call(
        _resid3_body,
        grid=(rows // bm,),
        in_specs=[_mat_spec(bm, D), _fix_spec((1, D)), _fix_spec((1, D)),
                  _mat_spec(bm, D), _fix_spec((D, D)), _fix_spec((D, D))],
        out_specs=[_mat_spec(bm, D)] * 3,
        out_shape=[jax.ShapeDtypeStruct((rows, D), F32)] * 3,
    )(z, a, c, h, wi, wj)


def _affine(st, count, g, bb):
    mu = st[0] / count
    var = st[1] / count - mu * mu
    a = g * lax.rsqrt(var + 1e-5)
    c = bb - mu * a
    return a.reshape(1, D), c.reshape(1, D)


def _sc_gather2_body(nchunks, taba_hbm, tabb_hbm, idxa_hbm, idxb_hbm,
                     outa_hbm, outb_hbm, idxa_v, idxb_v, bufa, bufb, gsa, gsb):
    wid = lax.axis_index("s") * _NC + lax.axis_index("c")
    nrows = 8 * nchunks
    row0 = wid * nrows
    pltpu.sync_copy(idxa_hbm.at[pl.ds(row0, nrows)], idxa_v)
    pltpu.sync_copy(idxb_hbm.at[pl.ds(row0, nrows)], idxb_v)

    def body(g, carry):
        ra = g * 4
        cpa = [pltpu.async_copy(taba_hbm.at[idxa_v.at[ra + j]],
                                bufa.at[pl.ds(j * 128, 128)], gsa)
               for j in range(4)]
        cpb = [pltpu.async_copy(tabb_hbm.at[idxb_v.at[ra + j]],
                                bufb.at[pl.ds(j * 128, 128)], gsb)
               for j in range(4)]
        for cp in cpa:
            cp.wait()
        pltpu.sync_copy(bufa, outa_hbm.at[pl.ds((row0 + ra) * 128, 512)])
        for cp in cpb:
            cp.wait()
        pltpu.sync_copy(bufb, outb_hbm.at[pl.ds((row0 + ra) * 128, 512)])
        return carry

    lax.fori_loop(0, 2 * nchunks, body, 0)


def _sc_gather2(taba, tabb, idxa2d, idxb2d, ep):
    nchunks = ep // (1024 * _NC * _NS)
    mesh = plsc.VectorSubcoreMesh(core_axis_name="c", subcore_axis_name="s")
    return pl.kernel(
        functools.partial(_sc_gather2_body, nchunks),
        mesh=mesh,
        out_type=[jax.ShapeDtypeStruct((ep, D), F32)] * 2,
        scratch_types=[pltpu.VMEM((8 * nchunks, 128), jnp.int32),
                       pltpu.VMEM((8 * nchunks, 128), jnp.int32),
                       pltpu.VMEM((512, D), F32),
                       pltpu.VMEM((512, D), F32),
                       pltpu.SemaphoreType.DMA,
                       pltpu.SemaphoreType.DMA],
        compiler_params=pltpu.CompilerParams(use_tc_tiling_on_sc=False),
    )(taba, tabb, idxa2d, idxb2d)


def _sc_scatter_body(nchunks, n_out, zh_hbm, idxs_hbm, zeros_hbm, ac_hbm,
                     out_hbm, idx_v, vals_v, ac_v, acc_sh):
    c = lax.axis_index("c")
    t = lax.axis_index("s")
    zrows = NACC // _NS
    pltpu.sync_copy(zeros_hbm.at[pl.ds(t * zrows, zrows)],
                    acc_sh.at[pl.ds(t * zrows, zrows)])
    pltpu.sync_copy(ac_hbm.at[0, c], ac_v.at[0])
    pltpu.sync_copy(ac_hbm.at[1, c], ac_v.at[1])
    a0 = ac_v[0, pl.ds(0, 16)]
    a1 = ac_v[0, pl.ds(16, 16)]
    c0 = ac_v[1, pl.ds(0, 16)]
    c1 = ac_v[1, pl.ds(16, 16)]
    plsc.subcore_barrier()
    row0 = t * (4 * nchunks)

    def body(s, carry):
        r0 = row0 + s * 4
        pltpu.sync_copy(idxs_hbm.at[pl.ds(r0, 4)], idx_v)
        pltpu.sync_copy(zh_hbm.at[c, pl.ds(r0 * 128, 512)], vals_v)

        def inner(r, cy):
            vals_v[r, pl.ds(0, 16)] = jnp.maximum(
                vals_v[r, pl.ds(0, 16)] * a0 + c0, 0.0)
            vals_v[r, pl.ds(16, 16)] = jnp.maximum(
                vals_v[r, pl.ds(16, 16)] * a1 + c1, 0.0)
            return cy

        lax.fori_loop(0, 512, inner, 0)
        for j in range(4):
            pltpu.sync_copy(vals_v.at[pl.ds(j * 128, 128)],
                            acc_sh.at[idx_v.at[j]], add=True)
        return carry

    lax.fori_loop(0, nchunks, body, 0)
    plsc.subcore_barrier()
    orow = n_out // _NS
    pltpu.sync_copy(acc_sh.at[pl.ds(t * orow, orow)],
                    out_hbm.at[c, pl.ds(t * orow, orow)])


def _sc_scatter(zh, idxs2d, zeros_acc, ac, n_out):
    ep = zh.shape[1]
    nchunks = ep // (512 * _NS)
    mesh = plsc.VectorSubcoreMesh(core_axis_name="c", subcore_axis_name="s")
    return pl.kernel(
        functools.partial(_sc_scatter_body, nchunks, n_out),
        mesh=mesh,
        out_type=jax.ShapeDtypeStruct((2, n_out, 32), F32),
        scratch_types=[pltpu.VMEM((4, 128), jnp.int32),
                       pltpu.VMEM((512, 32), F32),
                       pltpu.VMEM((2, 32), F32),
                       pltpu.VMEM_SHARED((NACC, 32), F32)],
        compiler_params=pltpu.CompilerParams(use_tc_tiling_on_sc=False),
    )(zh, idxs2d, zeros_acc, ac)


def kernel(x, edge_attr, edge_index, w_in, b_in, w_e, b_e,
           msg_w1, msg_b1, msg_g1, msg_bb1, msg_w2, msg_b2, msg_g2, msg_bb2,
           upd_w1, upd_b1, upd_g1, upd_bb1, upd_w2, upd_b2, upd_g2, upd_bb2,
           w_pred, b_pred):
    src = edge_index[0]
    dst = edge_index[1]
    n = x.shape[0]
    e_cnt = src.shape[0]
    pad = EP - e_cnt

    dst_g = jnp.concatenate([dst, jnp.zeros((pad,), jnp.int32)]).reshape(EP // 128, 128)
    src_g = jnp.concatenate([src, jnp.zeros((pad,), jnp.int32)]).reshape(EP // 128, 128)
    dst_s = jnp.concatenate([dst, jnp.full((pad,), n, jnp.int32)]).reshape(EP // 128, 128)
    zeros_acc = jnp.zeros((NACC, 32), F32)
    ea2d = jnp.concatenate([edge_attr,
                            jnp.zeros((pad,), F32)]).reshape(EP, 1)

    h, ha, hb = _proj3(x, w_in, b_in, msg_w1[0, :D], msg_w1[0, D:2 * D])
    e = None
    out = None
    pend = None
    for l in range(3):
        we = msg_w1[l, 2 * D:]
        ga, gb = _sc_gather2(ha, hb, dst_g, src_g, EP)
        if pend is not None:
            zp, ap, cp = pend
            if l == 1:
                e = _passCh0(zp, ap, cp, ea2d, w_e, b_e.reshape(1, D))
            else:
                e = _passCh(zp, ap, cp, e)
            pend = None
        if l == 0:
            v = w_e @ we
            z1, st1 = _passA0(ga, gb, ea2d, v, EP, e_cnt)
        else:
            z1, st1 = _passA(ga, gb, e, we, EP, e_cnt)
        a1, c1 = _affine(st1, e_cnt, msg_g1[l], msg_bb1[l])
        if l == 2:
            z2, st2 = _armm(z1, a1, c1, msg_w2[l], e_cnt)
            a2, c2 = _affine(st2, e_cnt, msg_g2[l], msg_bb2[l])
            out = _passCP(z2, a2, c2, e, w_pred.reshape(1, D), b_pred)
            break
        z2h, st2 = _armm_split(z1, a1, c1, msg_w2[l], e_cnt)
        a2, c2 = _affine(st2, e_cnt, msg_g2[l], msg_bb2[l])
        ac = jnp.concatenate([a2.reshape(1, 2, 32), c2.reshape(1, 2, 32)],
                             axis=0)
        aggh = _sc_scatter(z2h, dst_s, zeros_acc, ac, n)
        pend = (z2h, a2, c2)
        zu1, su1 = _nodeP1(h, aggh, upd_w1[l, :D], upd_w1[l, D:D + 32],
                           upd_w1[l, D + 32:])
        au1, cu1 = _affine(su1, n, upd_g1[l], upd_bb1[l])
        zu2, su2 = _armm(zu1, au1, cu1, upd_w2[l])
        au2, cu2 = _affine(su2, n, upd_g2[l], upd_bb2[l])
        h, ha, hb = _resid3(zu2, au2, cu2, h, msg_w1[l + 1, :D],
                            msg_w1[l + 1, D:2 * D])
    return out.reshape(-1)[:e_cnt, None]


# speedup vs baseline: 1.0018x; 1.0016x over previous
import functools

import jax
import jax.numpy as jnp
from jax import lax
from jax.experimental import pallas as pl
from jax.experimental.pallas import tpu as pltpu
from jax.experimental.pallas import tpu_sc as plsc

F32 = jnp.float32
D = 64
EP = 819200
NACC = 50016
_NC, _NS = 2, 16


def _acc_stats(st_ref, z, valid):
    ncol = z.shape[1]

    @pl.when(pl.program_id(0) == 0)
    def _():
        k0 = jnp.sum(z, axis=0) / z.shape[0]
        st_ref[...] = jnp.concatenate(
            [jnp.zeros((2, ncol), F32), k0[None, :],
             jnp.zeros((5, ncol), F32)], axis=0)

    k = st_ref[...][2][None, :]
    zc = z - k
    if valid is not None:
        bm = z.shape[0]
        row = (pl.program_id(0) * bm
               + lax.broadcasted_iota(jnp.int32, (bm, 1), 0))
        zc = jnp.where(row < valid, zc, 0.0)
    upd = jnp.concatenate(
        [jnp.sum(zc, axis=0)[None, :], jnp.sum(zc * zc, axis=0)[None, :],
         jnp.zeros((6, ncol), F32)], axis=0)
    st_ref[...] += upd


def _proj3_body(x_ref, w_ref, b_ref, wi_ref, wj_ref, h_ref, ha_ref, hb_ref):
    h = jnp.dot(x_ref[...], w_ref[...],
                preferred_element_type=F32) + b_ref[...][0][None, :]
    h_ref[...] = h
    ha_ref[...] = jnp.dot(h, wi_ref[...], preferred_element_type=F32)
    hb_ref[...] = jnp.dot(h, wj_ref[...], preferred_element_type=F32)


def _passA_body(valid, ga_ref, gb_ref, e_ref, w_ref, z_ref, st_ref):
    z = ga_ref[...] + gb_ref[...] + jnp.dot(e_ref[...], w_ref[...],
                                            preferred_element_type=F32)
    z_ref[...] = z
    _acc_stats(st_ref, z, valid)


def _passA0_body(valid, ga_ref, gb_ref, ea_ref, v_ref, z_ref, st_ref):
    z = ga_ref[...] + gb_ref[...] + ea_ref[...] * v_ref[...]
    z_ref[...] = z
    _acc_stats(st_ref, z, valid)


def _armm_body(valid, z_ref, a_ref, c_ref, w_ref, o_ref, st_ref):
    m = jnp.maximum(z_ref[...] * a_ref[...][0][None, :] + c_ref[...][0][None, :], 0.0)
    z2 = jnp.dot(m, w_ref[...], preferred_element_type=F32)
    o_ref[...] = z2
    _acc_stats(st_ref, z2, valid)


def _armm_split_body(valid, z_ref, a_ref, c_ref, w_ref, oh_ref, st_ref):
    m = jnp.maximum(z_ref[...] * a_ref[...][0][None, :] + c_ref[...][0][None, :], 0.0)
    z2 = jnp.dot(m, w_ref[...], preferred_element_type=F32)
    oh_ref[0, :, :] = z2[:, :32]
    oh_ref[1, :, :] = z2[:, 32:]
    _acc_stats(st_ref, z2, valid)


def _passCh_body(zh_ref, a_ref, c_ref, p_ref, enew_ref):
    z = jnp.concatenate([zh_ref[0], zh_ref[1]], axis=1)
    m2 = jnp.maximum(z * a_ref[...][0][None, :] + c_ref[...][0][None, :], 0.0)
    enew_ref[...] = p_ref[...] + m2


def _passCh0_body(zh_ref, a_ref, c_ref, ea_ref, wv_ref, bv_ref, enew_ref):
    z = jnp.concatenate([zh_ref[0], zh_ref[1]], axis=1)
    m2 = jnp.maximum(z * a_ref[...][0][None, :] + c_ref[...][0][None, :], 0.0)
    enew_ref[...] = ea_ref[...] * wv_ref[...] + bv_ref[...] + m2


def _passCP_body(z_ref, a_ref, c_ref, p_ref, w_ref, b_ref, o_ref):
    m2 = jnp.maximum(z_ref[...] * a_ref[...][0][None, :] + c_ref[...][0][None, :], 0.0)
    s = jnp.sum((p_ref[...] + m2) * w_ref[...][0][None, :], axis=1) + b_ref[...][0, 0]
    o_ref[...] = s.reshape(o_ref.shape)


def _nodeP1_body(h_ref, agg_ref, uh_ref, ua0_ref, ua1_ref, z_ref, st_ref):
    z = (jnp.dot(h_ref[...], uh_ref[...], preferred_element_type=F32)
         + jnp.dot(agg_ref[0], ua0_ref[...], preferred_element_type=F32)
         + jnp.dot(agg_ref[1], ua1_ref[...], preferred_element_type=F32))
    z_ref[...] = z
    _acc_stats(st_ref, z, None)


def _resid3_body(z_ref, a_ref, c_ref, h_ref, wi_ref, wj_ref,
                 o_ref, ha_ref, hb_ref):
    hn = h_ref[...] + jnp.maximum(
        z_ref[...] * a_ref[...][0][None, :] + c_ref[...][0][None, :], 0.0)
    o_ref[...] = hn
    ha_ref[...] = jnp.dot(hn, wi_ref[...], preferred_element_type=F32)
    hb_ref[...] = jnp.dot(hn, wj_ref[...], preferred_element_type=F32)


def _rows_block(rows):
    for bm in (4096, 5000):
        if rows % bm == 0:
            return bm
    return rows


def _mat_spec(bm, ncols):
    return pl.BlockSpec((bm, ncols), lambda i: (i, 0))


def _fix_spec(shape):
    nd = len(shape)
    return pl.BlockSpec(shape, lambda i: (0,) * nd)


def _half_spec(bm):
    return pl.BlockSpec((2, bm, 32), lambda i: (0, i, 0))


_ST_SPEC = pl.BlockSpec((8, D), lambda i: (0, 0))
_ST_SHAPE = jax.ShapeDtypeStruct((8, D), F32)


def _proj3(x, w, b, wi, wj):
    rows, k = x.shape
    bm = _rows_block(rows)
    return pl.pallas_call(
        _proj3_body,
        grid=(rows // bm,),
        in_specs=[_mat_spec(bm, k), _fix_spec((k, D)), _fix_spec((1, D)),
                  _fix_spec((D, D)), _fix_spec((D, D))],
        out_specs=[_mat_spec(bm, D)] * 3,
        out_shape=[jax.ShapeDtypeStruct((rows, D), F32)] * 3,
    )(x, w, b.reshape(1, D), wi, wj)


def _passA(ga, gb, e, w, rows, valid):
    bm = _rows_block(rows)
    return pl.pallas_call(
        functools.partial(_passA_body, valid),
        grid=(rows // bm,),
        in_specs=[_mat_spec(bm, D)] * 3 + [_fix_spec((D, D))],
        out_specs=[_mat_spec(bm, D), _ST_SPEC],
        out_shape=[jax.ShapeDtypeStruct((rows, D), F32), _ST_SHAPE],
    )(ga, gb, e, w)


def _passA0(ga, gb, ea2d, v, rows, valid):
    bm = _rows_block(rows)
    return pl.pallas_call(
        functools.partial(_passA0_body, valid),
        grid=(rows // bm,),
        in_specs=[_mat_spec(bm, D), _mat_spec(bm, D),
                  _mat_spec(bm, 1), _fix_spec((1, D))],
        out_specs=[_mat_spec(bm, D), _ST_SPEC],
        out_shape=[jax.ShapeDtypeStruct((rows, D), F32), _ST_SHAPE],
    )(ga, gb, ea2d, v)


def _armm(z, a, c, w, valid=None):
    rows = z.shape[0]
    bm = _rows_block(rows)
    return pl.pallas_call(
        functools.partial(_armm_body, valid),
        grid=(rows // bm,),
        in_specs=[_mat_spec(bm, D), _fix_spec((1, D)), _fix_spec((1, D)),
                  _fix_spec((D, D))],
        out_specs=[_mat_spec(bm, D), _ST_SPEC],
        out_shape=[jax.ShapeDtypeStruct((rows, D), F32), _ST_SHAPE],
    )(z, a, c, w)


def _armm_split(z, a, c, w, valid):
    rows = z.shape[0]
    bm = _rows_block(rows)
    return pl.pallas_call(
        functools.partial(_armm_split_body, valid),
        grid=(rows // bm,),
        in_specs=[_mat_spec(bm, D), _fix_spec((1, D)), _fix_spec((1, D)),
                  _fix_spec((D, D))],
        out_specs=[_half_spec(bm), _ST_SPEC],
        out_shape=[jax.ShapeDtypeStruct((2, rows, 32), F32), _ST_SHAPE],
    )(z, a, c, w)


def _passCh(zh, a, c, prev):
    rows = prev.shape[0]
    bm = _rows_block(rows)
    return pl.pallas_call(
        _passCh_body,
        grid=(rows // bm,),
        in_specs=[_half_spec(bm), _fix_spec((1, D)), _fix_spec((1, D)),
                  _mat_spec(bm, D)],
        out_specs=_mat_spec(bm, D),
        out_shape=jax.ShapeDtypeStruct((rows, D), F32),
    )(zh, a, c, prev)


def _passCh0(zh, a, c, ea2d, wv, bv):
    rows = zh.shape[1]
    bm = _rows_block(rows)
    return pl.pallas_call(
        _passCh0_body,
        grid=(rows // bm,),
        in_specs=[_half_spec(bm), _fix_spec((1, D)), _fix_spec((1, D)),
                  _mat_spec(bm, 1), _fix_spec((1, D)), _fix_spec((1, D))],
        out_specs=_mat_spec(bm, D),
        out_shape=jax.ShapeDtypeStruct((rows, D), F32),
    )(zh, a, c, ea2d, wv, bv)


def _passCP(z, a, c, prev, wt, b):
    rows = z.shape[0]
    bm = _rows_block(rows)
    return pl.pallas_call(
        _passCP_body,
        grid=(rows // bm,),
        in_specs=[_mat_spec(bm, D), _fix_spec((1, D)), _fix_spec((1, D)),
                  _mat_spec(bm, D), _fix_spec((1, D)), _fix_spec((1, 1))],
        out_specs=_mat_spec(bm // 128, 128),
        out_shape=jax.ShapeDtypeStruct((rows // 128, 128), F32),
    )(z, a, c, prev, wt, b.reshape(1, 1))


def _nodeP1(h, aggh, uh, ua0, ua1):
    rows = h.shape[0]
    bm = _rows_block(rows)
    return pl.pallas_call(
        _nodeP1_body,
        grid=(rows // bm,),
        in_specs=[_mat_spec(bm, D), _half_spec(bm),
                  _fix_spec((D, D)), _fix_spec((32, D)), _fix_spec((32, D))],
        out_specs=[_mat_spec(bm, D), _ST_SPEC],
        out_shape=[jax.ShapeDtypeStruct((rows, D), F32), _ST_SHAPE],
    )(h, aggh, uh, ua0, ua1)


def _resid3(z, a, c, h, wi, wj):
    rows = h.shape[0]
    bm = _rows_block(rows)
    return pl.pallas_call(
        _resid3_body,
        grid=(rows // bm,),
        in_specs=[_mat_spec(bm, D), _fix_spec((1, D)), _fix_spec((1, D)),
                  _mat_spec(bm, D), _fix_spec((D, D)), _fix_spec((D, D))],
        out_specs=[_mat_spec(bm, D)] * 3,
        out_shape=[jax.ShapeDtypeStruct((rows, D), F32)] * 3,
    )(z, a, c, h, wi, wj)


def _affine(st, count, g, bb):
    d1 = st[0] / count
    mu = d1 + st[2]
    var = st[1] / count - d1 * d1
    a = g * lax.rsqrt(var + 1e-5)
    c = bb - mu * a
    return a.reshape(1, D), c.reshape(1, D)


def _sc_gather2_body(nchunks, taba_hbm, tabb_hbm, idxa_hbm, idxb_hbm,
                     outa_hbm, outb_hbm, idxa_v, idxb_v, bufa, bufb, gsa, gsb):
    wid = lax.axis_index("s") * _NC + lax.axis_index("c")
    nrows = 8 * nchunks
    row0 = wid * nrows
    pltpu.sync_copy(idxa_hbm.at[pl.ds(row0, nrows)], idxa_v)
    pltpu.sync_copy(idxb_hbm.at[pl.ds(row0, nrows)], idxb_v)

    def body(g, carry):
        ra = g * 4
        cpa = [pltpu.async_copy(taba_hbm.at[idxa_v.at[ra + j]],
                                bufa.at[pl.ds(j * 128, 128)], gsa)
               for j in range(4)]
        cpb = [pltpu.async_copy(tabb_hbm.at[idxb_v.at[ra + j]],
                                bufb.at[pl.ds(j * 128, 128)], gsb)
               for j in range(4)]
        for cp in cpa:
            cp.wait()
        pltpu.sync_copy(bufa, outa_hbm.at[pl.ds((row0 + ra) * 128, 512)])
        for cp in cpb:
            cp.wait()
        pltpu.sync_copy(bufb, outb_hbm.at[pl.ds((row0 + ra) * 128, 512)])
        return carry

    lax.fori_loop(0, 2 * nchunks, body, 0)


def _sc_gather2(taba, tabb, idxa2d, idxb2d, ep):
    nchunks = ep // (1024 * _NC * _NS)
    mesh = plsc.VectorSubcoreMesh(core_axis_name="c", subcore_axis_name="s")
    return pl.kernel(
        functools.partial(_sc_gather2_body, nchunks),
        mesh=mesh,
        out_type=[jax.ShapeDtypeStruct((ep, D), F32)] * 2,
        scratch_types=[pltpu.VMEM((8 * nchunks, 128), jnp.int32),
                       pltpu.VMEM((8 * nchunks, 128), jnp.int32),
                       pltpu.VMEM((512, D), F32),
                       pltpu.VMEM((512, D), F32),
                       pltpu.SemaphoreType.DMA,
                       pltpu.SemaphoreType.DMA],
        compiler_params=pltpu.CompilerParams(use_tc_tiling_on_sc=False),
    )(taba, tabb, idxa2d, idxb2d)


def _sc_scatter_body(nchunks, n_out, zh_hbm, idxs_hbm, zeros_hbm, ac_hbm,
                     out_hbm, idx_v, vals_v, ac_v, acc_sh):
    c = lax.axis_index("c")
    t = lax.axis_index("s")
    zrows = NACC // _NS
    pltpu.sync_copy(zeros_hbm.at[pl.ds(t * zrows, zrows)],
                    acc_sh.at[pl.ds(t * zrows, zrows)])
    pltpu.sync_copy(ac_hbm.at[0, c], ac_v.at[0])
    pltpu.sync_copy(ac_hbm.at[1, c], ac_v.at[1])
    a0 = ac_v[0, pl.ds(0, 16)]
    a1 = ac_v[0, pl.ds(16, 16)]
    c0 = ac_v[1, pl.ds(0, 16)]
    c1 = ac_v[1, pl.ds(16, 16)]
    plsc.subcore_barrier()
    row0 = t * (4 * nchunks)

    def body(s, carry):
        r0 = row0 + s * 4
        pltpu.sync_copy(idxs_hbm.at[pl.ds(r0, 4)], idx_v)
        pltpu.sync_copy(zh_hbm.at[c, pl.ds(r0 * 128, 512)], vals_v)

        def inner(r, cy):
            vals_v[r, pl.ds(0, 16)] = jnp.maximum(
                vals_v[r, pl.ds(0, 16)] * a0 + c0, 0.0)
            vals_v[r, pl.ds(16, 16)] = jnp.maximum(
                vals_v[r, pl.ds(16, 16)] * a1 + c1, 0.0)
            return cy

        lax.fori_loop(0, 512, inner, 0)
        for j in range(4):
            pltpu.sync_copy(vals_v.at[pl.ds(j * 128, 128)],
                            acc_sh.at[idx_v.at[j]], add=True)
        return carry

    lax.fori_loop(0, nchunks, body, 0)
    plsc.subcore_barrier()
    orow = n_out // _NS
    pltpu.sync_copy(acc_sh.at[pl.ds(t * orow, orow)],
                    out_hbm.at[c, pl.ds(t * orow, orow)])


def _sc_scatter(zh, idxs2d, zeros_acc, ac, n_out):
    ep = zh.shape[1]
    nchunks = ep // (512 * _NS)
    mesh = plsc.VectorSubcoreMesh(core_axis_name="c", subcore_axis_name="s")
    return pl.kernel(
        functools.partial(_sc_scatter_body, nchunks, n_out),
        mesh=mesh,
        out_type=jax.ShapeDtypeStruct((2, n_out, 32), F32),
        scratch_types=[pltpu.VMEM((4, 128), jnp.int32),
                       pltpu.VMEM((512, 32), F32),
                       pltpu.VMEM((2, 32), F32),
                       pltpu.VMEM_SHARED((NACC, 32), F32)],
        compiler_params=pltpu.CompilerParams(use_tc_tiling_on_sc=False),
    )(zh, idxs2d, zeros_acc, ac)


def kernel(x, edge_attr, edge_index, w_in, b_in, w_e, b_e,
           msg_w1, msg_b1, msg_g1, msg_bb1, msg_w2, msg_b2, msg_g2, msg_bb2,
           upd_w1, upd_b1, upd_g1, upd_bb1, upd_w2, upd_b2, upd_g2, upd_bb2,
           w_pred, b_pred):
    src = edge_index[0]
    dst = edge_index[1]
    n = x.shape[0]
    e_cnt = src.shape[0]
    pad = EP - e_cnt

    dst_g = jnp.concatenate([dst, jnp.zeros((pad,), jnp.int32)]).reshape(EP // 128, 128)
    src_g = jnp.concatenate([src, jnp.zeros((pad,), jnp.int32)]).reshape(EP // 128, 128)
    dst_s = jnp.concatenate([dst, jnp.full((pad,), n, jnp.int32)]).reshape(EP // 128, 128)
    zeros_acc = jnp.zeros((NACC, 32), F32)
    ea2d = jnp.concatenate([edge_attr,
                            jnp.zeros((pad,), F32)]).reshape(EP, 1)

    h, ha, hb = _proj3(x, w_in, b_in, msg_w1[0, :D], msg_w1[0, D:2 * D])
    e = None
    out = None
    pend = None
    for l in range(3):
        we = msg_w1[l, 2 * D:]
        ga, gb = _sc_gather2(ha, hb, dst_g, src_g, EP)
        if pend is not None:
            zp, ap, cp = pend
            if l == 1:
                e = _passCh0(zp, ap, cp, ea2d, w_e, b_e.reshape(1, D))
            else:
                e = _passCh(zp, ap, cp, e)
            pend = None
        if l == 0:
            v = w_e @ we
            z1, st1 = _passA0(ga, gb, ea2d, v, EP, e_cnt)
        else:
            z1, st1 = _passA(ga, gb, e, we, EP, e_cnt)
        a1, c1 = _affine(st1, e_cnt, msg_g1[l], msg_bb1[l])
        if l == 2:
            z2, st2 = _armm(z1, a1, c1, msg_w2[l], e_cnt)
            a2, c2 = _affine(st2, e_cnt, msg_g2[l], msg_bb2[l])
            out = _passCP(z2, a2, c2, e, w_pred.reshape(1, D), b_pred)
            break
        z2h, st2 = _armm_split(z1, a1, c1, msg_w2[l], e_cnt)
        a2, c2 = _affine(st2, e_cnt, msg_g2[l], msg_bb2[l])
        ac = jnp.concatenate([a2.reshape(1, 2, 32), c2.reshape(1, 2, 32)],
                             axis=0)
        aggh = _sc_scatter(z2h, dst_s, zeros_acc, ac, n)
        pend = (z2h, a2, c2)
        zu1, su1 = _nodeP1(h, aggh, upd_w1[l, :D], upd_w1[l, D:D + 32],
                           upd_w1[l, D + 32:])
        au1, cu1 = _affine(su1, n, upd_g1[l], upd_bb1[l])
        zu2, su2 = _armm(zu1, au1, cu1, upd_w2[l])
        au2, cu2 = _affine(su2, n, upd_g2[l], upd_bb2[l])
        h, ha, hb = _resid3(zu2, au2, cu2, h, msg_w1[l + 1, :D],
                            msg_w1[l + 1, D:2 * D])
    return out.reshape(-1)[:e_cnt, None]


# speedup vs baseline: 1.0052x; 1.0034x over previous
import functools

import jax
import jax.numpy as jnp
from jax import lax
from jax.experimental import pallas as pl
from jax.experimental.pallas import tpu as pltpu
from jax.experimental.pallas import tpu_sc as plsc

F32 = jnp.float32
D = 64
EP = 819200
NACC = 50016
_NC, _NS = 2, 16


def _acc_stats(st_ref, z, valid):
    ncol = z.shape[1]

    @pl.when(pl.program_id(0) == 0)
    def _():
        k0 = jnp.sum(z, axis=0) / z.shape[0]
        st_ref[...] = jnp.concatenate(
            [jnp.zeros((2, ncol), F32), k0[None, :],
             jnp.zeros((5, ncol), F32)], axis=0)

    k = st_ref[...][2][None, :]
    zc = z - k
    if valid is not None:
        bm = z.shape[0]
        row = (pl.program_id(0) * bm
               + lax.broadcasted_iota(jnp.int32, (bm, 1), 0))
        zc = jnp.where(row < valid, zc, 0.0)
    upd = jnp.concatenate(
        [jnp.sum(zc, axis=0)[None, :], jnp.sum(zc * zc, axis=0)[None, :],
         jnp.zeros((6, ncol), F32)], axis=0)
    st_ref[...] += upd


def _proj3_body(x_ref, w_ref, b_ref, wi_ref, wj_ref, h_ref, ha_ref, hb_ref):
    h = jnp.dot(x_ref[...], w_ref[...],
                preferred_element_type=F32) + b_ref[...][0][None, :]
    h_ref[...] = h
    ha_ref[...] = jnp.dot(h, wi_ref[...], preferred_element_type=F32)
    hb_ref[...] = jnp.dot(h, wj_ref[...], preferred_element_type=F32)


def _passA_body(valid, ga_ref, gb_ref, e_ref, w_ref, z_ref, st_ref):
    z = ga_ref[...] + gb_ref[...] + jnp.dot(e_ref[...], w_ref[...],
                                            preferred_element_type=F32)
    z_ref[...] = z
    _acc_stats(st_ref, z, valid)


def _passA0_body(valid, ga_ref, gb_ref, ea_ref, v_ref, z_ref, st_ref):
    z = ga_ref[...] + gb_ref[...] + ea_ref[...] * v_ref[...]
    z_ref[...] = z
    _acc_stats(st_ref, z, valid)


def _armm_body(valid, z_ref, a_ref, c_ref, w_ref, o_ref, st_ref):
    m = jnp.maximum(z_ref[...] * a_ref[...][0][None, :] + c_ref[...][0][None, :], 0.0)
    z2 = jnp.dot(m, w_ref[...], preferred_element_type=F32)
    o_ref[...] = z2
    _acc_stats(st_ref, z2, valid)


def _armm_split_body(valid, z_ref, a_ref, c_ref, w_ref, oh_ref, st_ref):
    m = jnp.maximum(z_ref[...] * a_ref[...][0][None, :] + c_ref[...][0][None, :], 0.0)
    z2 = jnp.dot(m, w_ref[...], preferred_element_type=F32)
    oh_ref[0, :, :] = z2[:, :32]
    oh_ref[1, :, :] = z2[:, 32:]
    _acc_stats(st_ref, z2, valid)


def _passCh_body(zh_ref, a_ref, c_ref, p_ref, enew_ref):
    z = jnp.concatenate([zh_ref[0], zh_ref[1]], axis=1)
    m2 = jnp.maximum(z * a_ref[...][0][None, :] + c_ref[...][0][None, :], 0.0)
    enew_ref[...] = p_ref[...] + m2


def _passCh0_body(zh_ref, a_ref, c_ref, ea_ref, wv_ref, bv_ref, enew_ref):
    z = jnp.concatenate([zh_ref[0], zh_ref[1]], axis=1)
    m2 = jnp.maximum(z * a_ref[...][0][None, :] + c_ref[...][0][None, :], 0.0)
    enew_ref[...] = ea_ref[...] * wv_ref[...] + bv_ref[...] + m2


def _passCP_body(z_ref, a_ref, c_ref, p_ref, w_ref, b_ref, o_ref):
    m2 = jnp.maximum(z_ref[...] * a_ref[...][0][None, :] + c_ref[...][0][None, :], 0.0)
    s = jnp.sum((p_ref[...] + m2) * w_ref[...][0][None, :], axis=1) + b_ref[...][0, 0]
    o_ref[...] = s.reshape(o_ref.shape)


def _nodeP1_body(h_ref, agg_ref, uh_ref, ua0_ref, ua1_ref, z_ref, st_ref):
    z = (jnp.dot(h_ref[...], uh_ref[...], preferred_element_type=F32)
         + jnp.dot(agg_ref[0], ua0_ref[...], preferred_element_type=F32)
         + jnp.dot(agg_ref[1], ua1_ref[...], preferred_element_type=F32))
    z_ref[...] = z
    _acc_stats(st_ref, z, None)


def _resid3_body(z_ref, a_ref, c_ref, h_ref, wi_ref, wj_ref,
                 o_ref, ha_ref, hb_ref):
    hn = h_ref[...] + jnp.maximum(
        z_ref[...] * a_ref[...][0][None, :] + c_ref[...][0][None, :], 0.0)
    o_ref[...] = hn
    ha_ref[...] = jnp.dot(hn, wi_ref[...], preferred_element_type=F32)
    hb_ref[...] = jnp.dot(hn, wj_ref[...], preferred_element_type=F32)


def _rows_block(rows):
    for bm in (4096, 5000):
        if rows % bm == 0:
            return bm
    return rows


def _mat_spec(bm, ncols):
    return pl.BlockSpec((bm, ncols), lambda i: (i, 0))


def _fix_spec(shape):
    nd = len(shape)
    return pl.BlockSpec(shape, lambda i: (0,) * nd)


def _half_spec(bm):
    return pl.BlockSpec((2, bm, 32), lambda i: (0, i, 0))


_ST_SPEC = pl.BlockSpec((8, D), lambda i: (0, 0))
_ST_SHAPE = jax.ShapeDtypeStruct((8, D), F32)


def _proj3(x, w, b, wi, wj):
    rows, k = x.shape
    bm = _rows_block(rows)
    return pl.pallas_call(
        _proj3_body,
        grid=(rows // bm,),
        in_specs=[_mat_spec(bm, k), _fix_spec((k, D)), _fix_spec((1, D)),
                  _fix_spec((D, D)), _fix_spec((D, D))],
        out_specs=[_mat_spec(bm, D)] * 3,
        out_shape=[jax.ShapeDtypeStruct((rows, D), F32)] * 3,
    )(x, w, b.reshape(1, D), wi, wj)


def _passA(ga, gb, e, w, rows, valid):
    bm = _rows_block(rows)
    return pl.pallas_call(
        functools.partial(_passA_body, valid),
        grid=(rows // bm,),
        in_specs=[_mat_spec(bm, D)] * 3 + [_fix_spec((D, D))],
        out_specs=[_mat_spec(bm, D), _ST_SPEC],
        out_shape=[jax.ShapeDtypeStruct((rows, D), F32), _ST_SHAPE],
    )(ga, gb, e, w)


def _passA0(ga, gb, ea2d, v, rows, valid):
    bm = _rows_block(rows)
    return pl.pallas_call(
        functools.partial(_passA0_body, valid),
        grid=(rows // bm,),
        in_specs=[_mat_spec(bm, D), _mat_spec(bm, D),
                  _mat_spec(bm, 1), _fix_spec((1, D))],
        out_specs=[_mat_spec(bm, D), _ST_SPEC],
        out_shape=[jax.ShapeDtypeStruct((rows, D), F32), _ST_SHAPE],
    )(ga, gb, ea2d, v)


def _armm(z, a, c, w, valid=None):
    rows = z.shape[0]
    bm = _rows_block(rows)
    return pl.pallas_call(
        functools.partial(_armm_body, valid),
        grid=(rows // bm,),
        in_specs=[_mat_spec(bm, D), _fix_spec((1, D)), _fix_spec((1, D)),
                  _fix_spec((D, D))],
        out_specs=[_mat_spec(bm, D), _ST_SPEC],
        out_shape=[jax.ShapeDtypeStruct((rows, D), F32), _ST_SHAPE],
    )(z, a, c, w)


def _armm_split(z, a, c, w, valid):
    rows = z.shape[0]
    bm = _rows_block(rows)
    return pl.pallas_call(
        functools.partial(_armm_split_body, valid),
        grid=(rows // bm,),
        in_specs=[_mat_spec(bm, D), _fix_spec((1, D)), _fix_spec((1, D)),
                  _fix_spec((D, D))],
        out_specs=[_half_spec(bm), _ST_SPEC],
        out_shape=[jax.ShapeDtypeStruct((2, rows, 32), F32), _ST_SHAPE],
    )(z, a, c, w)


def _passCh(zh, a, c, prev):
    rows = prev.shape[0]
    bm = _rows_block(rows)
    return pl.pallas_call(
        _passCh_body,
        grid=(rows // bm,),
        in_specs=[_half_spec(bm), _fix_spec((1, D)), _fix_spec((1, D)),
                  _mat_spec(bm, D)],
        out_specs=_mat_spec(bm, D),
        out_shape=jax.ShapeDtypeStruct((rows, D), F32),
    )(zh, a, c, prev)


def _passCh0(zh, a, c, ea2d, wv, bv):
    rows = zh.shape[1]
    bm = _rows_block(rows)
    return pl.pallas_call(
        _passCh0_body,
        grid=(rows // bm,),
        in_specs=[_half_spec(bm), _fix_spec((1, D)), _fix_spec((1, D)),
                  _mat_spec(bm, 1), _fix_spec((1, D)), _fix_spec((1, D))],
        out_specs=_mat_spec(bm, D),
        out_shape=jax.ShapeDtypeStruct((rows, D), F32),
    )(zh, a, c, ea2d, wv, bv)


def _passCP(z, a, c, prev, wt, b):
    rows = z.shape[0]
    bm = _rows_block(rows)
    return pl.pallas_call(
        _passCP_body,
        grid=(rows // bm,),
        in_specs=[_mat_spec(bm, D), _fix_spec((1, D)), _fix_spec((1, D)),
                  _mat_spec(bm, D), _fix_spec((1, D)), _fix_spec((1, 1))],
        out_specs=_mat_spec(bm // 128, 128),
        out_shape=jax.ShapeDtypeStruct((rows // 128, 128), F32),
    )(z, a, c, prev, wt, b.reshape(1, 1))


def _nodeP1(h, aggh, uh, ua0, ua1):
    rows = h.shape[0]
    bm = _rows_block(rows)
    return pl.pallas_call(
        _nodeP1_body,
        grid=(rows // bm,),
        in_specs=[_mat_spec(bm, D), _half_spec(bm),
                  _fix_spec((D, D)), _fix_spec((32, D)), _fix_spec((32, D))],
        out_specs=[_mat_spec(bm, D), _ST_SPEC],
        out_shape=[jax.ShapeDtypeStruct((rows, D), F32), _ST_SHAPE],
    )(h, aggh, uh, ua0, ua1)


def _resid3(z, a, c, h, wi, wj):
    rows = h.shape[0]
    bm = _rows_block(rows)
    return pl.pallas_call(
        _resid3_body,
        grid=(rows // bm,),
        in_specs=[_mat_spec(bm, D), _fix_spec((1, D)), _fix_spec((1, D)),
                  _mat_spec(bm, D), _fix_spec((D, D)), _fix_spec((D, D))],
        out_specs=[_mat_spec(bm, D)] * 3,
        out_shape=[jax.ShapeDtypeStruct((rows, D), F32)] * 3,
    )(z, a, c, h, wi, wj)


def _affine(st, count, g, bb):
    d1 = st[0] / count
    mu = d1 + st[2]
    var = st[1] / count - d1 * d1
    a = g * lax.rsqrt(var + 1e-5)
    c = bb - mu * a
    return a.reshape(1, D), c.reshape(1, D)


def _sc_gather2_body(nchunks, taba_hbm, tabb_hbm, idxa_hbm, idxb_hbm,
                     outa_hbm, outb_hbm, idxa_v, idxb_v, bufa, bufb, gsa, gsb):
    wid = lax.axis_index("s") * _NC + lax.axis_index("c")
    nrows = 8 * nchunks
    row0 = wid * nrows
    pltpu.sync_copy(idxa_hbm.at[pl.ds(row0, nrows)], idxa_v)
    pltpu.sync_copy(idxb_hbm.at[pl.ds(row0, nrows)], idxb_v)

    def body(g, carry):
        ra = g * 4
        cpa = [pltpu.async_copy(taba_hbm.at[idxa_v.at[ra + j]],
                                bufa.at[pl.ds(j * 128, 128)], gsa)
               for j in range(4)]
        cpb = [pltpu.async_copy(tabb_hbm.at[idxb_v.at[ra + j]],
                                bufb.at[pl.ds(j * 128, 128)], gsb)
               for j in range(4)]
        for cp in cpa:
            cp.wait()
        pltpu.sync_copy(bufa, outa_hbm.at[pl.ds((row0 + ra) * 128, 512)])
        for cp in cpb:
            cp.wait()
        pltpu.sync_copy(bufb, outb_hbm.at[pl.ds((row0 + ra) * 128, 512)])
        return carry

    lax.fori_loop(0, 2 * nchunks, body, 0)


def _sc_gather2(taba, tabb, idxa2d, idxb2d, ep):
    nchunks = ep // (1024 * _NC * _NS)
    mesh = plsc.VectorSubcoreMesh(core_axis_name="c", subcore_axis_name="s")
    return pl.kernel(
        functools.partial(_sc_gather2_body, nchunks),
        mesh=mesh,
        out_type=[jax.ShapeDtypeStruct((ep, D), F32)] * 2,
        scratch_types=[pltpu.VMEM((8 * nchunks, 128), jnp.int32),
                       pltpu.VMEM((8 * nchunks, 128), jnp.int32),
                       pltpu.VMEM((512, D), F32),
                       pltpu.VMEM((512, D), F32),
                       pltpu.SemaphoreType.DMA,
                       pltpu.SemaphoreType.DMA],
        compiler_params=pltpu.CompilerParams(use_tc_tiling_on_sc=False),
    )(taba, tabb, idxa2d, idxb2d)


def _sc_scatter_body(nchunks, n_out, zh_hbm, idxs_hbm, zeros_hbm, ac_hbm,
                     out_hbm, idx_v, vals_v, ac_v, acc_sh, asem):
    c = lax.axis_index("c")
    t = lax.axis_index("s")
    zrows = NACC // _NS
    pltpu.sync_copy(zeros_hbm.at[pl.ds(t * zrows, zrows)],
                    acc_sh.at[pl.ds(t * zrows, zrows)])
    pltpu.sync_copy(ac_hbm.at[0, c], ac_v.at[0])
    pltpu.sync_copy(ac_hbm.at[1, c], ac_v.at[1])
    a0 = ac_v[0, pl.ds(0, 16)]
    a1 = ac_v[0, pl.ds(16, 16)]
    c0 = ac_v[1, pl.ds(0, 16)]
    c1 = ac_v[1, pl.ds(16, 16)]
    plsc.subcore_barrier()
    row0 = t * (4 * nchunks)

    def body(s, carry):
        r0 = row0 + s * 4
        pltpu.sync_copy(idxs_hbm.at[pl.ds(r0, 4)], idx_v)
        pltpu.sync_copy(zh_hbm.at[c, pl.ds(r0 * 128, 512)], vals_v)

        def inner(r, cy):
            vals_v[r, pl.ds(0, 16)] = jnp.maximum(
                vals_v[r, pl.ds(0, 16)] * a0 + c0, 0.0)
            vals_v[r, pl.ds(16, 16)] = jnp.maximum(
                vals_v[r, pl.ds(16, 16)] * a1 + c1, 0.0)
            return cy

        lax.fori_loop(0, 512, inner, 0)
        cps = [pltpu.async_copy(vals_v.at[pl.ds(j * 128, 128)],
                                acc_sh.at[idx_v.at[j]], asem, add=True)
               for j in range(4)]
        for cp in cps:
            cp.wait()
        return carry

    lax.fori_loop(0, nchunks, body, 0)
    plsc.subcore_barrier()
    orow = n_out // _NS
    pltpu.sync_copy(acc_sh.at[pl.ds(t * orow, orow)],
                    out_hbm.at[c, pl.ds(t * orow, orow)])


def _sc_scatter(zh, idxs2d, zeros_acc, ac, n_out):
    ep = zh.shape[1]
    nchunks = ep // (512 * _NS)
    mesh = plsc.VectorSubcoreMesh(core_axis_name="c", subcore_axis_name="s")
    return pl.kernel(
        functools.partial(_sc_scatter_body, nchunks, n_out),
        mesh=mesh,
        out_type=jax.ShapeDtypeStruct((2, n_out, 32), F32),
        scratch_types=[pltpu.VMEM((4, 128), jnp.int32),
                       pltpu.VMEM((512, 32), F32),
                       pltpu.VMEM((2, 32), F32),
                       pltpu.VMEM_SHARED((NACC, 32), F32),
                       pltpu.SemaphoreType.DMA],
        compiler_params=pltpu.CompilerParams(use_tc_tiling_on_sc=False),
    )(zh, idxs2d, zeros_acc, ac)


def kernel(x, edge_attr, edge_index, w_in, b_in, w_e, b_e,
           msg_w1, msg_b1, msg_g1, msg_bb1, msg_w2, msg_b2, msg_g2, msg_bb2,
           upd_w1, upd_b1, upd_g1, upd_bb1, upd_w2, upd_b2, upd_g2, upd_bb2,
           w_pred, b_pred):
    src = edge_index[0]
    dst = edge_index[1]
    n = x.shape[0]
    e_cnt = src.shape[0]
    pad = EP - e_cnt

    dst_g = jnp.concatenate([dst, jnp.zeros((pad,), jnp.int32)]).reshape(EP // 128, 128)
    src_g = jnp.concatenate([src, jnp.zeros((pad,), jnp.int32)]).reshape(EP // 128, 128)
    dst_s = jnp.concatenate([dst, jnp.full((pad,), n, jnp.int32)]).reshape(EP // 128, 128)
    zeros_acc = jnp.zeros((NACC, 32), F32)
    ea2d = jnp.concatenate([edge_attr,
                            jnp.zeros((pad,), F32)]).reshape(EP, 1)

    h, ha, hb = _proj3(x, w_in, b_in, msg_w1[0, :D], msg_w1[0, D:2 * D])
    e = None
    out = None
    pend = None
    for l in range(3):
        we = msg_w1[l, 2 * D:]
        ga, gb = _sc_gather2(ha, hb, dst_g, src_g, EP)
        if pend is not None:
            zp, ap, cp = pend
            if l == 1:
                e = _passCh0(zp, ap, cp, ea2d, w_e, b_e.reshape(1, D))
            else:
                e = _passCh(zp, ap, cp, e)
            pend = None
        if l == 0:
            v = w_e @ we
            z1, st1 = _passA0(ga, gb, ea2d, v, EP, e_cnt)
        else:
            z1, st1 = _passA(ga, gb, e, we, EP, e_cnt)
        a1, c1 = _affine(st1, e_cnt, msg_g1[l], msg_bb1[l])
        if l == 2:
            z2, st2 = _armm(z1, a1, c1, msg_w2[l], e_cnt)
            a2, c2 = _affine(st2, e_cnt, msg_g2[l], msg_bb2[l])
            out = _passCP(z2, a2, c2, e, w_pred.reshape(1, D), b_pred)
            break
        z2h, st2 = _armm_split(z1, a1, c1, msg_w2[l], e_cnt)
        a2, c2 = _affine(st2, e_cnt, msg_g2[l], msg_bb2[l])
        ac = jnp.concatenate([a2.reshape(1, 2, 32), c2.reshape(1, 2, 32)],
                             axis=0)
        aggh = _sc_scatter(z2h, dst_s, zeros_acc, ac, n)
        pend = (z2h, a2, c2)
        zu1, su1 = _nodeP1(h, aggh, upd_w1[l, :D], upd_w1[l, D:D + 32],
                           upd_w1[l, D + 32:])
        au1, cu1 = _affine(su1, n, upd_g1[l], upd_bb1[l])
        zu2, su2 = _armm(zu1, au1, cu1, upd_w2[l])
        au2, cu2 = _affine(su2, n, upd_g2[l], upd_bb2[l])
        h, ha, hb = _resid3(zu2, au2, cu2, h, msg_w1[l + 1, :D],
                            msg_w1[l + 1, D:2 * D])
    return out.reshape(-1)[:e_cnt, None]
